# Initial kernel scaffold; baseline (speedup 1.0000x reference)
import jax
import jax.numpy as jnp
from jax.experimental import pallas as pl


def kernel(inputs, targets):
    raise NotImplementedError("write your pallas kernel here")


# speedup vs baseline: 1.2117x; 1.2117x over previous
import jax
import jax.numpy as jnp
from jax.experimental import pallas as pl
from jax.experimental.pallas import tpu as pltpu

_MARGIN = 0.3
_N = 32
_F = 2048
_L = 64
_HALF = _N // 2
_SCALE = 1.0 / float(_F * _L)


def _sort_body(x_ref, o_ref):
    x = x_ref[...]
    li = jax.lax.broadcasted_iota(jnp.int32, x.shape, 2)
    for k in (2, 4, 8, 16, 32, 64):
        if k < 64:
            dir_up = (li & k) == 0
        j = k // 2
        while j >= 1:
            is_low = (li & j) == 0
            if k == 64:
                take_min = is_low
            else:
                take_min = dir_up == is_low
            rm = pltpu.roll(x, 128 - j, 2)
            rp = pltpu.roll(x, j, 2)
            partner = jnp.where(is_low, rm, rp)
            mn = jnp.minimum(x, partner)
            mx = jnp.maximum(x, partner)
            x = jnp.where(take_min, mn, mx)
            j //= 2
    o_ref[...] = x


def _dist_loss_body(s_ref, t_ref, o_ref, dist_ref):
    c = pl.program_id(0)
    i0 = c * _HALF
    other = (1 - c) * _HALF

    def pair_sum(a, b):
        x = s_ref[a]
        y = s_ref[b]
        return jnp.sum(jnp.abs(x - y)) * _SCALE

    def inner_within(j, i):
        d = pair_sum(i0 + i, i0 + j)
        dist_ref[i, i0 + j] = d
        dist_ref[j, i0 + i] = d
        return i

    def outer_within(i, _):
        jax.lax.fori_loop(i + 1, _HALF, inner_within, i)
        dist_ref[i, i0 + i] = 0.0
        return 0

    jax.lax.fori_loop(0, _HALF, outer_within, 0)

    def inner_cross(jo, i):
        dist_ref[i, other + jo] = pair_sum(i0 + i, other + jo)
        return i

    def outer_cross(i, _):
        jax.lax.fori_loop(0, _HALF, inner_cross, i)
        return 0

    jax.lax.fori_loop(0, _HALF, outer_cross, 0)

    def loss_row(i, acc):
        ti = t_ref[i0 + i]

        def scan_j(j, carry):
            ap, an = carry
            d = dist_ref[i, j]
            same = t_ref[j] == ti
            ap = jnp.where(same, jnp.maximum(ap, d), ap)
            an = jnp.where(same, an, jnp.minimum(an, d))
            return ap, an

        ap, an = jax.lax.fori_loop(
            0, _N, scan_j, (jnp.float32(-jnp.inf), jnp.float32(jnp.inf))
        )
        return acc + jnp.maximum(ap - an + _MARGIN, 0.0)

    total = jax.lax.fori_loop(0, _HALF, loss_row, jnp.float32(0.0))
    o_ref[...] = jnp.full((1, 8, 128), total, jnp.float32)


def kernel(inputs, targets):
    x = inputs.reshape(_N, _F * _L // 128, 128)
    cf = _F * _L // 128 // 1

    s = pl.pallas_call(
        _sort_body,
        grid=(2, _HALF),
        in_specs=[
            pl.BlockSpec((1, cf, 128), lambda c, i: (c * _HALF + i, 0, 0))
        ],
        out_specs=pl.BlockSpec((1, cf, 128), lambda c, i: (c * _HALF + i, 0, 0)),
        out_shape=jax.ShapeDtypeStruct((_N, cf, 128), jnp.float32),
        compiler_params=pltpu.CompilerParams(
            dimension_semantics=("parallel", "arbitrary"),
            vmem_limit_bytes=64 * 1024 * 1024,
        ),
    )(x)

    partial = pl.pallas_call(
        _dist_loss_body,
        grid=(2,),
        in_specs=[
            pl.BlockSpec((_N, cf, 128), lambda c: (0, 0, 0)),
            pl.BlockSpec(memory_space=pltpu.SMEM),
        ],
        out_specs=pl.BlockSpec((1, 8, 128), lambda c: (c, 0, 0)),
        out_shape=jax.ShapeDtypeStruct((2, 8, 128), jnp.float32),
        scratch_shapes=[pltpu.SMEM((_HALF, _N), jnp.float32)],
        compiler_params=pltpu.CompilerParams(
            dimension_semantics=("parallel",),
            vmem_limit_bytes=48 * 1024 * 1024,
        ),
    )(s, targets.astype(jnp.int32))

    return (partial[0, 0, 0] + partial[1, 0, 0]) / jnp.float32(_N)


# speedup vs baseline: 1.7748x; 1.4647x over previous
import jax
import jax.numpy as jnp
from jax.experimental import pallas as pl
from jax.experimental.pallas import tpu as pltpu

_MARGIN = 0.3
_N = 32
_F = 2048
_L = 64
_HALF = _N // 2
_SCALE = 1.0 / float(_F * _L)


def _sort_body(x_ref, o_ref):
    x = x_ref[...]
    li = jax.lax.broadcasted_iota(jnp.int32, x.shape, 2)
    for k in (2, 4, 8, 16, 32, 64):
        if k < 64:
            dir_up = (li & k) == 0
        j = k // 2
        while j >= 1:
            is_low = (li & j) == 0
            if k == 64:
                take_min = is_low
            else:
                take_min = dir_up == is_low
            rm = pltpu.roll(x, 128 - j, 2)
            rp = pltpu.roll(x, j, 2)
            partner = jnp.where(is_low, rm, rp)
            mn = jnp.minimum(x, partner)
            mx = jnp.maximum(x, partner)
            x = jnp.where(take_min, mn, mx)
            j //= 2
    o_ref[...] = x


def _dist_loss_body(s_ref, t_ref, tv_ref, o_ref):
    c = pl.program_id(0)
    i0 = c * _HALF
    tv = tv_ref[...]

    def loss_row(i, acc):
        row = s_ref[i0 + i]
        ti = t_ref[i0 + i]
        ap = jnp.full((1, 8), -jnp.inf, jnp.float32)
        an = jnp.full((1, 8), jnp.inf, jnp.float32)
        for jb in range(_N // 8):
            chunk = s_ref[jb * 8 : (jb + 1) * 8]
            d8 = jnp.sum(jnp.abs(chunk - row[None, :, :]), axis=(1, 2))
            d8 = (d8 * _SCALE).reshape(1, 8)
            m8 = tv[:, jb * 8 : (jb + 1) * 8] == ti
            ap = jnp.maximum(ap, jnp.where(m8, d8, -jnp.inf))
            an = jnp.minimum(an, jnp.where(m8, jnp.inf, d8))
        hp = jnp.max(ap)
        hn = jnp.min(an)
        return acc + jnp.maximum(hp - hn + _MARGIN, 0.0)

    total = jax.lax.fori_loop(0, _HALF, loss_row, jnp.float32(0.0))
    o_ref[...] = jnp.full((1, 8, 128), total, jnp.float32)


def kernel(inputs, targets):
    x = inputs.reshape(_N, _F * _L // 128, 128)
    cf = _F * _L // 128 // 1

    s = pl.pallas_call(
        _sort_body,
        grid=(2, _HALF),
        in_specs=[
            pl.BlockSpec((1, cf, 128), lambda c, i: (c * _HALF + i, 0, 0))
        ],
        out_specs=pl.BlockSpec((1, cf, 128), lambda c, i: (c * _HALF + i, 0, 0)),
        out_shape=jax.ShapeDtypeStruct((_N, cf, 128), jnp.float32),
        compiler_params=pltpu.CompilerParams(
            dimension_semantics=("parallel", "arbitrary"),
            vmem_limit_bytes=64 * 1024 * 1024,
        ),
    )(x)

    partial = pl.pallas_call(
        _dist_loss_body,
        grid=(2,),
        in_specs=[
            pl.BlockSpec((_N, cf, 128), lambda c: (0, 0, 0)),
            pl.BlockSpec(memory_space=pltpu.SMEM),
            pl.BlockSpec((1, _N), lambda c: (0, 0)),
        ],
        out_specs=pl.BlockSpec((1, 8, 128), lambda c: (c, 0, 0)),
        out_shape=jax.ShapeDtypeStruct((2, 8, 128), jnp.float32),
        compiler_params=pltpu.CompilerParams(
            dimension_semantics=("parallel",),
            vmem_limit_bytes=48 * 1024 * 1024,
        ),
    )(s, targets.astype(jnp.int32), targets.astype(jnp.int32).reshape(1, _N))

    return (partial[0, 0, 0] + partial[1, 0, 0]) / jnp.float32(_N)
